# baseline (speedup 1.0000x reference)
import functools
import jax
import jax.numpy as jnp
from jax.experimental import pallas as pl

_G = 8
_AP = 96


def _vgnn_kernel(x_ref, na_ref,
                 wblk_ref, bblk_ref,
                 we_ref, be_ref,
                 wem_ref, bem_ref,
                 wpz_ref, bpz_ref,
                 wri_ref, wrh_ref, brn_ref,
                 wp1_ref, bp1_ref,
                 ws1_ref, wn1_ref, b1_ref,
                 wp2_ref, bp2_ref,
                 ws2_ref, wn2_ref, b2_ref,
                 out_ref, *, a_real):
    G = x_ref.shape[0]
    A = x_ref.shape[1]
    AP = _AP
    hid = we_ref.shape[1]
    T = bblk_ref.shape[1] // hid
    rows = G * AP

    we = we_ref[...]
    be = be_ref[...]
    wem = wem_ref[...]
    bem = bem_ref[...]
    wpz = wpz_ref[...]
    bpz = bpz_ref[...]
    wri = wri_ref[...]
    wrh = wrh_ref[...]
    brn = brn_ref[...]

    tf = x_ref.shape[2]
    xb = jnp.concatenate(
        [x_ref[...], jnp.zeros((G, AP - A, tf), jnp.float32)],
        axis=1).reshape(rows, tf)
    phi_all = jax.nn.relu(jnp.dot(xb, wblk_ref[...]) + bblk_ref[...])

    h = jnp.zeros((rows, hid), jnp.float32)
    pz = h
    for t in range(T):
        phi_x = jax.lax.slice_in_dim(phi_all, t * hid, (t + 1) * hid, axis=1)
        enc_h = jax.nn.relu(
            jnp.dot(jnp.concatenate([phi_x, h], axis=1), we) + be)
        z = jnp.dot(enc_h, wem) + bem
        pz = jax.nn.relu(jnp.dot(z, wpz) + bpz)
        h = jnp.tanh(
            jnp.dot(jnp.concatenate([phi_x, pz], axis=1), wri)
            + jnp.dot(h, wrh) + brn)

    arow = jax.lax.broadcasted_iota(jnp.int32, (G, AP, 1), 1)
    valid = arow < a_real

    def neighbor_max(m2d):
        feat = m2d.shape[1]
        m = m2d.reshape(G, AP, feat)
        mneg = jnp.where(valid, m, -jnp.inf)
        m1 = jnp.max(mneg, axis=1, keepdims=True)
        ismax = mneg == m1
        cnt = jnp.sum(ismax.astype(jnp.float32), axis=1, keepdims=True)
        m2 = jnp.max(jnp.where(ismax, -jnp.inf, mneg), axis=1, keepdims=True)
        nb = jnp.where(ismax & (cnt < 1.5), m2, m1)
        return nb.reshape(rows, feat)

    hn = jnp.concatenate([h, pz], axis=1)
    m1 = jax.nn.relu(jnp.dot(hn, wp1_ref[...]) + bp1_ref[...])
    nb1 = neighbor_max(m1)
    r1 = jnp.tanh(jnp.dot(hn, ws1_ref[...])
                  + jnp.dot(nb1, wn1_ref[...]) + b1_ref[...])

    m2 = jax.nn.relu(jnp.dot(r1, wp2_ref[...]) + bp2_ref[...])
    nb2 = neighbor_max(m2)
    r2 = (jnp.dot(r1, ws2_ref[...]) + jnp.dot(nb2, wn2_ref[...])
          + b2_ref[...])

    gh = r2.shape[1]
    r2m = jnp.where(valid, r2.reshape(G, AP, gh), 0.0)
    pooled = jnp.sum(r2m, axis=1) / na_ref[0, 0]
    out_ref[...] = pooled


@jax.jit
def kernel(agent_obs, hideout_obs, timestep_obs, num_agents, params):
    B, T, A, F = agent_obs.shape
    p = params
    hid = p['W_phi_x'].shape[1]
    gh = p['W_self2'].shape[1]

    def row(b):
        return b.reshape(1, -1)

    xt = jnp.transpose(agent_obs, (0, 2, 1, 3)).reshape(B, A, T * F)
    wblk = jnp.zeros((T * F, T * hid), jnp.float32)
    for t in range(T):
        wblk = jax.lax.dynamic_update_slice(
            wblk, p['W_phi_x'], (t * F, t * hid))
    bblk = jnp.tile(p['b_phi_x'], (T,)).reshape(1, T * hid)

    na = num_agents[:1].reshape(1, 1).astype(jnp.float32)
    operands = [
        xt, na,
        wblk, bblk,
        p['W_enc'], row(p['b_enc']),
        p['W_enc_mean'], row(p['b_enc_mean']),
        p['W_phi_z'], row(p['b_phi_z']),
        p['W_rnn_in'], p['W_rnn_h'], row(p['b_rnn']),
        p['W_pool1'], row(p['b_pool1']),
        p['W_self1'], p['W_neigh1'], row(p['b1']),
        p['W_pool2'], row(p['b_pool2']),
        p['W_self2'], p['W_neigh2'], row(p['b2']),
    ]

    in_specs = [pl.BlockSpec((_G, A, T * F), lambda i: (i, 0, 0))]
    for op in operands[1:]:
        in_specs.append(
            pl.BlockSpec(op.shape, lambda i, nd=op.ndim: (0,) * nd))

    pooled = pl.pallas_call(
        functools.partial(_vgnn_kernel, a_real=A),
        grid=(B // _G,),
        in_specs=in_specs,
        out_specs=pl.BlockSpec((_G, gh), lambda i: (i, 0)),
        out_shape=jax.ShapeDtypeStruct((B, gh), jnp.float32),
    )(*operands)

    return jnp.concatenate(
        [pooled, hideout_obs, timestep_obs], axis=-1)


# speedup vs baseline: 1.1712x; 1.1712x over previous
import functools
import jax
import jax.numpy as jnp
from jax.experimental import pallas as pl
from jax.experimental.pallas import tpu as pltpu

_G = 8
_AP = 96


def _vgnn_kernel(x_ref, na_ref,
                 wpx_ref, bpx_ref,
                 we_ref, be_ref,
                 wem_ref, bem_ref,
                 wpz_ref, bpz_ref,
                 wri_ref, wrh_ref, brn_ref,
                 wp1_ref, bp1_ref,
                 ws1_ref, wn1_ref, b1_ref,
                 wp2_ref, bp2_ref,
                 ws2_ref, wn2_ref, b2_ref,
                 out_ref, *, a_real):
    G = x_ref.shape[0]
    T = x_ref.shape[1]
    A = x_ref.shape[2]
    F = x_ref.shape[3]
    AP = _AP
    hid = wpx_ref.shape[1]
    rows = G * AP

    wpx = wpx_ref[...]
    bpx = bpx_ref[...]
    we = we_ref[...]
    be = be_ref[...]
    wem = wem_ref[...]
    bem = bem_ref[...]
    wpz = wpz_ref[...]
    bpz = bpz_ref[...]
    wri = wri_ref[...]
    wrh = wrh_ref[...]
    brn = brn_ref[...]

    zpad = jnp.zeros((G, AP - A, F), jnp.float32)

    h = jnp.zeros((rows, hid), jnp.float32)
    pz = h
    for t in range(T):
        x_t = jnp.concatenate([x_ref[:, t], zpad], axis=1).reshape(rows, F)
        phi_x = jax.nn.relu(jnp.dot(x_t, wpx) + bpx)
        enc_h = jax.nn.relu(
            jnp.dot(jnp.concatenate([phi_x, h], axis=1), we) + be)
        z = jnp.dot(enc_h, wem) + bem
        pz = jax.nn.relu(jnp.dot(z, wpz) + bpz)
        h = jnp.tanh(
            jnp.dot(jnp.concatenate([phi_x, pz], axis=1), wri)
            + jnp.dot(h, wrh) + brn)

    arow = jax.lax.broadcasted_iota(jnp.int32, (G, AP, 1), 1)
    valid = arow < a_real

    def neighbor_max(m2d):
        feat = m2d.shape[1]
        m = m2d.reshape(G, AP, feat)
        mneg = jnp.where(valid, m, -jnp.inf)
        m1 = jnp.max(mneg, axis=1, keepdims=True)
        ismax = mneg == m1
        cnt = jnp.sum(ismax.astype(jnp.float32), axis=1, keepdims=True)
        m2 = jnp.max(jnp.where(ismax, -jnp.inf, mneg), axis=1, keepdims=True)
        nb = jnp.where(ismax & (cnt < 1.5), m2, m1)
        return nb.reshape(rows, feat)

    hn = jnp.concatenate([h, pz], axis=1)
    m1 = jax.nn.relu(jnp.dot(hn, wp1_ref[...]) + bp1_ref[...])
    nb1 = neighbor_max(m1)
    r1 = jnp.tanh(jnp.dot(hn, ws1_ref[...])
                  + jnp.dot(nb1, wn1_ref[...]) + b1_ref[...])

    m2 = jax.nn.relu(jnp.dot(r1, wp2_ref[...]) + bp2_ref[...])
    nb2 = neighbor_max(m2)
    r2 = (jnp.dot(r1, ws2_ref[...]) + jnp.dot(nb2, wn2_ref[...])
          + b2_ref[...])

    gh = r2.shape[1]
    r2m = jnp.where(valid, r2.reshape(G, AP, gh), 0.0)
    pooled = jnp.sum(r2m, axis=1) / na_ref[0, 0]
    out_ref[...] = pooled


@jax.jit
def kernel(agent_obs, hideout_obs, timestep_obs, num_agents, params):
    B, T, A, F = agent_obs.shape
    p = params
    hid = p['W_phi_x'].shape[1]
    gh = p['W_self2'].shape[1]

    def row(b):
        return b.reshape(1, -1)

    na = num_agents[:1].reshape(1, 1).astype(jnp.float32)
    operands = [
        agent_obs, na,
        p['W_phi_x'], row(p['b_phi_x']),
        p['W_enc'], row(p['b_enc']),
        p['W_enc_mean'], row(p['b_enc_mean']),
        p['W_phi_z'], row(p['b_phi_z']),
        p['W_rnn_in'], p['W_rnn_h'], row(p['b_rnn']),
        p['W_pool1'], row(p['b_pool1']),
        p['W_self1'], p['W_neigh1'], row(p['b1']),
        p['W_pool2'], row(p['b_pool2']),
        p['W_self2'], p['W_neigh2'], row(p['b2']),
    ]

    in_specs = [pl.BlockSpec((_G, T, A, F), lambda i: (i, 0, 0, 0))]
    for op in operands[1:]:
        in_specs.append(
            pl.BlockSpec(op.shape, lambda i, nd=op.ndim: (0,) * nd))

    pooled = pl.pallas_call(
        functools.partial(_vgnn_kernel, a_real=A),
        grid=(B // _G,),
        in_specs=in_specs,
        out_specs=pl.BlockSpec((_G, gh), lambda i: (i, 0)),
        out_shape=jax.ShapeDtypeStruct((B, gh), jnp.float32),
        compiler_params=pltpu.CompilerParams(
            dimension_semantics=("parallel",)),
    )(*operands)

    return jnp.concatenate(
        [pooled, hideout_obs, timestep_obs], axis=-1)
